# Initial kernel scaffold; baseline (speedup 1.0000x reference)
import jax
import jax.numpy as jnp
from jax.experimental import pallas as pl


def kernel(x, s, edge_index, gcn_params, gna_params):
    raise NotImplementedError("write your pallas kernel here")


# speedup vs baseline: 8.8555x; 8.8555x over previous
import functools

import jax
import jax.numpy as jnp
from jax import lax
from jax.experimental import pallas as pl
from jax.experimental.pallas import tpu as pltpu
from jax.experimental.pallas import tpu_sc as plsc

NC = 2
NS = 16
EB = 128
ZR = 128
BN = 1000


def _mesh():
    return plsc.VectorSubcoreMesh(core_axis_name="c", subcore_axis_name="s")


def _make_deg_kernel(n_acc, e_pad):
    nw = NC * NS
    nblk = e_pad // (nw * EB)
    rpt = n_acc // NS

    @functools.partial(
        pl.kernel,
        out_type=jax.ShapeDtypeStruct((NC, n_acc, 16), jnp.float32),
        mesh=_mesh(),
        compiler_params=pltpu.CompilerParams(use_tc_tiling_on_sc=False),
        scratch_types=[
            pltpu.VMEM_SHARED((n_acc, 16), jnp.float32),
            pltpu.VMEM((ZR, 16), jnp.float32),
            pltpu.VMEM((EB, 16), jnp.float32),
            pltpu.VMEM((EB,), jnp.int32),
        ],
    )
    def k(dst_hbm, out_hbm, acc_sh, z_v, ones_v, dst_v):
        cid = lax.axis_index("c")
        sid = lax.axis_index("s")
        wid = sid * NC + cid
        zero = jnp.zeros((16,), jnp.float32)
        one = jnp.ones((16,), jnp.float32)

        @pl.loop(0, ZR)
        def _(j):
            z_v[j, :] = zero

        @pl.loop(0, EB)
        def _(j):
            ones_v[j, :] = one

        r0 = sid * rpt

        @pl.loop(0, rpt // ZR)
        def _(i):
            pltpu.sync_copy(z_v, acc_sh.at[pl.ds(r0 + i * ZR, ZR)])

        plsc.subcore_barrier()
        e0 = wid * nblk * EB

        @pl.loop(0, nblk)
        def _(i):
            pltpu.sync_copy(dst_hbm.at[pl.ds(e0 + i * EB, EB)], dst_v)
            pltpu.sync_copy(ones_v, acc_sh.at[dst_v], add=True)

        plsc.subcore_barrier()
        pltpu.sync_copy(acc_sh.at[pl.ds(r0, rpt)],
                        out_hbm.at[cid, pl.ds(r0, rpt)])

    return k


def _make_edge_kernel(n_acc, dx, e_pad):
    nw = NC * NS
    nblk = e_pad // (nw * EB)
    rpt = n_acc // NS

    @functools.partial(
        pl.kernel,
        out_type=(
            jax.ShapeDtypeStruct((NC, n_acc, dx), jnp.float32),
            jax.ShapeDtypeStruct((NC, n_acc, 16), jnp.float32),
        ),
        mesh=_mesh(),
        compiler_params=pltpu.CompilerParams(use_tc_tiling_on_sc=False),
        scratch_types=[
            pltpu.VMEM_SHARED((n_acc, dx), jnp.float32),
            pltpu.VMEM_SHARED((n_acc, 16), jnp.float32),
            pltpu.VMEM((EB,), jnp.int32),
            pltpu.VMEM((EB,), jnp.int32),
            pltpu.VMEM((EB, dx), jnp.float32),
            pltpu.VMEM((EB, 16), jnp.float32),
            pltpu.VMEM((EB, 16), jnp.float32),
            pltpu.VMEM((EB, 16), jnp.float32),
            pltpu.VMEM((16,), jnp.float32),
            pltpu.SemaphoreType.DMA,
        ],
    )
    def k(hw_hbm, m_hbm, src_hbm, dst_hbm, a_hbm, outx_hbm, outs_hbm,
          accx_sh, accs_sh, src_v, dst_v, rowsx_v, ms_v, md_v,
          o_v, a_v, sem):
        cid = lax.axis_index("c")
        sid = lax.axis_index("s")
        wid = sid * NC + cid
        zero = jnp.zeros((16,), jnp.float32)

        @pl.loop(0, ZR)
        def _(j):
            for t in range(dx // 16):
                rowsx_v[j, pl.ds(t * 16, 16)] = zero
            o_v[j, :] = zero

        pltpu.sync_copy(a_hbm, a_v)
        r0 = sid * rpt

        @pl.loop(0, rpt // ZR)
        def _(i):
            pltpu.sync_copy(rowsx_v, accx_sh.at[pl.ds(r0 + i * ZR, ZR)])
            pltpu.sync_copy(o_v, accs_sh.at[pl.ds(r0 + i * ZR, ZR)])

        plsc.subcore_barrier()
        e0 = wid * nblk * EB

        @pl.loop(0, nblk)
        def _(i):
            b = e0 + i * EB
            pltpu.sync_copy(src_hbm.at[pl.ds(b, EB)], src_v)
            pltpu.sync_copy(dst_hbm.at[pl.ds(b, EB)], dst_v)
            pltpu.async_copy(hw_hbm.at[src_v], rowsx_v, sem).wait()
            pltpu.async_copy(m_hbm.at[src_v], ms_v, sem).wait()
            pltpu.async_copy(m_hbm.at[dst_v], md_v, sem).wait()
            av = a_v[:]

            @pl.loop(0, EB)
            def _(j):
                ms = ms_v[j, :]
                md = md_v[j, :]
                t = (md - ms) * av
                al = 1.0 / (1.0 + jnp.exp(-t))
                o_v[j, :] = al * ms

            pltpu.sync_copy(rowsx_v, accx_sh.at[dst_v], add=True)
            pltpu.sync_copy(o_v, accs_sh.at[dst_v], add=True)

        plsc.subcore_barrier()
        pltpu.sync_copy(accx_sh.at[pl.ds(r0, rpt)],
                        outx_hbm.at[cid, pl.ds(r0, rpt)])
        pltpu.sync_copy(accs_sh.at[pl.ds(r0, rpt)],
                        outs_hbm.at[cid, pl.ds(r0, rpt)])

    return k


def _rspec(d):
    return pl.BlockSpec((BN, d), lambda i: (i, 0))


def _bspec(shape):
    return pl.BlockSpec(shape, lambda i: tuple(0 for _ in shape))


def _dinv_of(deg_ref):
    deg = deg_ref[:, 0] + deg_ref[:, 1] + 1.0
    return lax.rsqrt(deg)[:, None]


def _tc_pre(deg2, x, s, w0, w2p, b2p, w1p, b1p):
    n, dxi = x.shape
    dxo = w0.shape[1]

    def body(deg_ref, x_ref, s_ref, w0_ref, w2_ref, b2_ref, w1_ref, b1_ref,
             hw_ref, m_ref, gw1_ref):
        dinv = _dinv_of(deg_ref)
        hw_ref[...] = jnp.dot(x_ref[...], w0_ref[...],
                              preferred_element_type=jnp.float32) * dinv
        sv = s_ref[...]
        m_ref[...] = jnp.dot(sv, w2_ref[...],
                             preferred_element_type=jnp.float32) + b2_ref[...]
        gw1_ref[...] = jnp.dot(sv, w1_ref[...],
                               preferred_element_type=jnp.float32) + b1_ref[...]

    return pl.pallas_call(
        body,
        grid=(n // BN,),
        in_specs=[_rspec(2), _rspec(dxi),
                  _rspec(16), _bspec((dxi, dxo)), _bspec((16, 16)),
                  _bspec((1, 16)), _bspec((16, 16)), _bspec((1, 16))],
        out_specs=[_rspec(dxo), _rspec(16), _rspec(16)],
        out_shape=[jax.ShapeDtypeStruct((n, dxo), jnp.float32),
                   jax.ShapeDtypeStruct((n, 16), jnp.float32),
                   jax.ShapeDtypeStruct((n, 16), jnp.float32)],
    )(deg2, x, s, w0, w2p, b2p, w1p, b1p)


def _tc_mid(deg2, accx2, accs2, hwp, gw1p, bxp, w, w2p, b2p, w1p, b1p):
    n, dprev = hwp.shape
    dxo = w.shape[1]

    def body(deg_ref, ax_ref, as_ref, hwp_ref, gw1p_ref, bx_ref, w_ref,
             w2_ref, b2_ref, w1_ref, b1_ref, hw_ref, m_ref, gw1_ref):
        dinv = _dinv_of(deg_ref)
        h = jnp.maximum(
            dinv * (ax_ref[0] + ax_ref[1] + hwp_ref[...]) + bx_ref[...], 0.0)
        hw_ref[...] = jnp.dot(h, w_ref[...],
                              preferred_element_type=jnp.float32) * dinv
        g = jnp.maximum(gw1p_ref[...] + as_ref[0] + as_ref[1], 0.0)
        m_ref[...] = jnp.dot(g, w2_ref[...],
                             preferred_element_type=jnp.float32) + b2_ref[...]
        gw1_ref[...] = jnp.dot(g, w1_ref[...],
                               preferred_element_type=jnp.float32) + b1_ref[...]

    return pl.pallas_call(
        body,
        grid=(n // BN,),
        in_specs=[_rspec(2),
                  pl.BlockSpec((2, BN, dprev), lambda i: (0, i, 0)),
                  pl.BlockSpec((2, BN, 16), lambda i: (0, i, 0)),
                  _rspec(dprev), _rspec(16), _bspec((1, dprev)),
                  _bspec((dprev, dxo)), _bspec((16, 16)), _bspec((1, 16)),
                  _bspec((16, 16)), _bspec((1, 16))],
        out_specs=[_rspec(dxo), _rspec(16), _rspec(16)],
        out_shape=[jax.ShapeDtypeStruct((n, dxo), jnp.float32),
                   jax.ShapeDtypeStruct((n, 16), jnp.float32),
                   jax.ShapeDtypeStruct((n, 16), jnp.float32)],
    )(deg2, accx2, accs2, hwp, gw1p, bxp, w, w2p, b2p, w1p, b1p)


def _tc_final(deg2, accx2, accs2, hwp, gw1p, bxp):
    n, dprev = hwp.shape

    def body(deg_ref, ax_ref, as_ref, hwp_ref, gw1p_ref, bx_ref,
             xo_ref, so_ref):
        dinv = _dinv_of(deg_ref)
        xo_ref[...] = dinv * (ax_ref[0] + ax_ref[1] + hwp_ref[...]) + bx_ref[...]
        so_ref[...] = gw1p_ref[...] + as_ref[0] + as_ref[1]

    return pl.pallas_call(
        body,
        grid=(n // BN,),
        in_specs=[_rspec(2),
                  pl.BlockSpec((2, BN, dprev), lambda i: (0, i, 0)),
                  pl.BlockSpec((2, BN, 16), lambda i: (0, i, 0)),
                  _rspec(dprev), _rspec(16), _bspec((1, dprev))],
        out_specs=[_rspec(dprev), _rspec(16)],
        out_shape=[jax.ShapeDtypeStruct((n, dprev), jnp.float32),
                   jax.ShapeDtypeStruct((n, 16), jnp.float32)],
    )(deg2, accx2, accs2, hwp, gw1p, bxp)


def _pad16(w):
    out = jnp.zeros((16, 16), jnp.float32)
    return out.at[: w.shape[0], : w.shape[1]].set(w)


def _padv(v):
    out = jnp.zeros((1, 16), jnp.float32)
    return out.at[0, : v.shape[0]].set(v)


def kernel(x, s, edge_index, gcn_params, gna_params):
    n = x.shape[0]
    e = edge_index.shape[1]
    nw = NC * NS
    src = edge_index[0].astype(jnp.int32)
    dst = edge_index[1].astype(jnp.int32)
    n_acc = -(-(n + 1) // (NS * ZR)) * (NS * ZR)
    nblk = -(-e // (nw * EB))
    e_pad = nblk * nw * EB
    src_p = jnp.concatenate([src, jnp.zeros((e_pad - e,), jnp.int32)])
    dst_p = jnp.concatenate([dst, jnp.full((e_pad - e,), n, jnp.int32)])

    deg_part = _make_deg_kernel(n_acc, e_pad)(dst_p)
    deg2 = deg_part[:, :n, 0].T

    w1ps, b1ps, w2ps, b2ps, aps = [], [], [], [], []
    for (w1, b1, w2, b2, a) in gna_params:
        w1ps.append(_pad16(w1))
        b1ps.append(_padv(b1))
        w2ps.append(_pad16(w2))
        b2ps.append(_padv(b2))
        aps.append(_padv(a)[0])
    bxs = [p[1][None, :] for p in gcn_params]

    hw, m, gw1 = _tc_pre(deg2, x, s, gcn_params[0][0], w2ps[0], b2ps[0],
                         w1ps[0], b1ps[0])
    x_ = s_ = None
    for i in range(len(gcn_params)):
        accx, accs = _make_edge_kernel(n_acc, hw.shape[1], e_pad)(
            hw, m, src_p, dst_p, aps[i])
        accx2 = accx[:, :n]
        accs2 = accs[:, :n]
        if i < len(gcn_params) - 1:
            hw, m, gw1 = _tc_mid(deg2, accx2, accs2, hw, gw1, bxs[i],
                                 gcn_params[i + 1][0], w2ps[i + 1],
                                 b2ps[i + 1], w1ps[i + 1], b1ps[i + 1])
        else:
            x_, s_ = _tc_final(deg2, accx2, accs2, hw, gw1, bxs[i])
    return (x_, s_)
